# baseline (speedup 1.0000x reference)
import jax
import jax.numpy as jnp
from jax import lax
from jax.experimental import pallas as pl

NUM_KNOTS = 16
DOMAIN_MIN = -1.0
DOMAIN_MAX = 1.0
STEP = (DOMAIN_MAX - DOMAIN_MIN) / (NUM_KNOTS - 1)


def _body(x_ref, o_ref):
    x = x_ref[:]
    c = jnp.minimum(jnp.maximum(x, DOMAIN_MIN), DOMAIN_MAX)
    s = (c - DOMAIN_MIN) / STEP
    k = lax.broadcasted_iota(jnp.int32, o_ref.shape, 1).astype(jnp.float32)
    o_ref[:, :] = jnp.maximum(1.0 - jnp.abs(s - k), 0.0)


def kernel(inputs):
    n = inputs.shape[0]
    block = 2000
    grid = n // block
    return pl.pallas_call(
        _body,
        grid=(grid,),
        in_specs=[pl.BlockSpec((block, 1), lambda i: (i, 0))],
        out_specs=pl.BlockSpec((block, NUM_KNOTS), lambda i: (i, 0)),
        out_shape=jax.ShapeDtypeStruct((n, NUM_KNOTS), jnp.float32),
    )(inputs.reshape(n, 1))


# speedup vs baseline: 1.9910x; 1.9910x over previous
import jax
import jax.numpy as jnp
from jax import lax
from jax.experimental import pallas as pl

NUM_KNOTS = 16
DOMAIN_MIN = -1.0
DOMAIN_MAX = 1.0
STEP = (DOMAIN_MAX - DOMAIN_MIN) / (NUM_KNOTS - 1)


def _body(x_ref, o_ref):
    r = x_ref.shape[0]
    a = lax.broadcasted_iota(jnp.int32, (8, 128), 0)
    p = lax.broadcasted_iota(jnp.int32, (8, 128), 1)
    s_mat = (a == (p >> 4)).astype(jnp.float32)
    x_rep = lax.dot_general(
        x_ref[:, :], s_mat, (((1,), (0,)), ((), ())),
        preferred_element_type=jnp.float32,
        precision=lax.Precision.HIGHEST,
    )
    c = jnp.minimum(jnp.maximum(x_rep, DOMAIN_MIN), DOMAIN_MAX)
    s = (c - DOMAIN_MIN) / STEP
    k = (lax.broadcasted_iota(jnp.int32, (r, 128), 1) & 15).astype(jnp.float32)
    o_ref[:, :] = jnp.maximum(1.0 - jnp.abs(s - k), 0.0)


def kernel(inputs):
    n = inputs.shape[0]
    rows = n // 8
    block = 2000
    grid = rows // block
    out = pl.pallas_call(
        _body,
        grid=(grid,),
        in_specs=[pl.BlockSpec((block, 8), lambda i: (i, 0))],
        out_specs=pl.BlockSpec((block, 128), lambda i: (i, 0)),
        out_shape=jax.ShapeDtypeStruct((rows, 128), jnp.float32),
    )(inputs.reshape(rows, 8))
    return out.reshape(n, NUM_KNOTS)


# speedup vs baseline: 2.3493x; 1.1800x over previous
import functools

import jax
import jax.numpy as jnp
from jax import lax
from jax.experimental import pallas as pl
from jax.experimental.pallas import tpu as pltpu
from jax.experimental.pallas import tpu_sc as plsc

NUM_KNOTS = 16
DOMAIN_MIN = -1.0
DOMAIN_MAX = 1.0
STEP = (DOMAIN_MAX - DOMAIN_MIN) / (NUM_KNOTS - 1)
INV_STEP = 7.5

NC = 2
NS = 16
NW = NC * NS

E = 2000


def _compute_chunk(in_buf, out_buf):
    lanes = lax.broadcasted_iota(jnp.int32, (16,), 0)
    zero = jnp.zeros((16,), jnp.float32)

    def step(i, carry):
        x = in_buf[pl.ds(i * 16, 16)]
        c = jnp.minimum(jnp.maximum(x, DOMAIN_MIN), DOMAIN_MAX)
        s = (c - DOMAIN_MIN) * INV_STEP
        left = s.astype(jnp.int32)
        frac = s - left.astype(jnp.float32)
        right = jnp.minimum(left + 1, NUM_KNOTS - 1)
        base = i * (16 * NUM_KNOTS) + lanes * NUM_KNOTS
        for jj in range(16):
            out_buf[pl.ds(i * (16 * NUM_KNOTS) + jj * 16, 16)] = zero
        plsc.store_scatter(out_buf, [base + left], 1.0 - frac)
        plsc.addupdate_scatter(out_buf, [base + right], frac)
        return carry

    lax.fori_loop(0, E // 16, step, 0, unroll=2)


def _sc_body(in_hbm, out_hbm, in_buf0, in_buf1, out_buf0, out_buf1, in_sem0,
             in_sem1, out_sem0, out_sem1):
    n = in_hbm.shape[0]
    n_chunks = n // E
    wid = lax.axis_index("s") * NC + lax.axis_index("c")
    jmax = (n_chunks + NW - 1) // NW
    in_bufs = (in_buf0, in_buf1)
    out_bufs = (out_buf0, out_buf1)
    in_sems = (in_sem0, in_sem1)
    out_sems = (out_sem0, out_sem1)

    def chunk_id(j):
        return wid + NW * j

    def in_copy(j):
        b = j % 2
        return pltpu.make_async_copy(
            in_hbm.at[pl.ds(chunk_id(j) * E, E)], in_bufs[b], in_sems[b])

    def out_copy(j):
        b = j % 2
        return pltpu.make_async_copy(
            out_bufs[b],
            out_hbm.at[pl.ds(chunk_id(j) * E * NUM_KNOTS, E * NUM_KNOTS)],
            out_sems[b])

    def guarded(j, fn):
        @pl.when(chunk_id(j) < n_chunks)
        def _():
            fn()

    guarded(0, lambda: in_copy(0).start())
    for j in range(jmax):
        if j + 1 < jmax:
            guarded(j + 1, lambda j=j: in_copy(j + 1).start())
        guarded(j, lambda j=j: in_copy(j).wait())
        if j >= 2:
            guarded(j, lambda j=j: out_copy(j - 2).wait())
        guarded(j, lambda j=j: _compute_chunk(in_bufs[j % 2], out_bufs[j % 2]))
        guarded(j, lambda j=j: out_copy(j).start())
    for j in (jmax - 2, jmax - 1):
        if j >= 0:
            guarded(j, lambda j=j: out_copy(j).wait())


def kernel(inputs):
    n = inputs.shape[0]
    sc_kernel = functools.partial(
        pl.kernel,
        out_type=jax.ShapeDtypeStruct((n * NUM_KNOTS,), jnp.float32),
        mesh=plsc.VectorSubcoreMesh(core_axis_name="c", subcore_axis_name="s"),
        compiler_params=pltpu.CompilerParams(needs_layout_passes=False),
        scratch_types=[
            pltpu.VMEM((E,), jnp.float32),
            pltpu.VMEM((E,), jnp.float32),
            pltpu.VMEM((E * NUM_KNOTS,), jnp.float32),
            pltpu.VMEM((E * NUM_KNOTS,), jnp.float32),
            pltpu.SemaphoreType.DMA,
            pltpu.SemaphoreType.DMA,
            pltpu.SemaphoreType.DMA,
            pltpu.SemaphoreType.DMA,
        ],
    )(_sc_body)
    return sc_kernel(inputs).reshape(n, NUM_KNOTS)
